# baseline (device time: 109038 ns/iter reference)
import jax
import jax.numpy as jnp
from jax import lax
from jax.experimental import pallas as pl
from jax.experimental.pallas import tpu as pltpu

RQ = 1024
CHUNKS = ((0, 64), (64, 192), (256, 256), (512, 256), (768, 256))
NC = len(CHUNKS)
A = 360
B = 336
C2 = RQ - A - B

def _pieces(lo, hi):
    out = {}
    idx = NC
    for k, (off, rows) in enumerate(CHUNKS):
        p_lo, p_hi = max(off, lo), min(off + rows, hi)
        if p_lo < p_hi:
            out[k] = (p_lo, p_hi - p_lo, idx)
            idx += 1
    return out

Y2P = _pieces(A, A + B)
Z2P = _pieces(A + B, RQ)


def kernel(x):
    m, n = x.shape
    half = n // 2
    out_m = 2 * m

    def body(x_ref, out_ref, sbuf, xrbuf, yrbuf, zrbuf, keep_buf,
             keep_sem, keep_out_sem, stage_sems,
             xsend, xrecv, ysend, yrecv, zsend, zrecv, csem):
        my_x = lax.axis_index("x")
        my_y = lax.axis_index("y")
        my_z = lax.axis_index("z")
        xp = (1 - my_x, my_y, my_z)
        yp = (my_x, 1 - my_y, my_z)
        zp = (my_x, my_y, 1 - my_z)
        q = 2 * my_y + my_z
        qy = 2 * (1 - my_y) + my_z
        qz = 2 * my_y + (1 - my_z)
        qd = 2 * (1 - my_y) + (1 - my_z)

        ocol = (1 - my_x) * half
        obase = my_x * m
        rbase = (1 - my_x) * m

        def rdma(src, dst, ssem, rsem, dev):
            return pltpu.make_async_remote_copy(
                src_ref=src, dst_ref=dst, send_sem=ssem, recv_sem=rsem,
                device_id=dev, device_id_type=pl.DeviceIdType.MESH,
            )

        barrier = pltpu.get_barrier_semaphore()
        for nb in (xp, yp, zp):
            pl.semaphore_signal(
                barrier, inc=1, device_id=nb,
                device_id_type=pl.DeviceIdType.MESH,
            )
        pl.semaphore_wait(barrier, 3)

        keep_in = pltpu.make_async_copy(
            x_ref.at[:, pl.ds(my_x * half, half)], keep_buf, keep_sem,
        )
        keep_in.start()

        st0 = pltpu.make_async_copy(
            x_ref.at[pl.ds(q * RQ, RQ), pl.ds(ocol, half)],
            sbuf.at[pl.ds(0, RQ), :], stage_sems.at[0],
        )
        st1 = pltpu.make_async_copy(
            x_ref.at[pl.ds(qd * RQ, A), pl.ds(ocol, half)],
            sbuf.at[pl.ds(RQ, A), :], stage_sems.at[1],
        )
        st0.start()
        st1.start()
        st0.wait()

        xd = []
        for k, (off, rows) in enumerate(CHUNKS):
            d = rdma(sbuf.at[pl.ds(off, rows), :],
                     xrbuf.at[pl.ds(off, rows), :],
                     xsend.at[k], xrecv.at[k], xp)
            d.start()
            xd.append(d)
        st1.wait()
        xe = rdma(sbuf.at[pl.ds(RQ, A), :],
                  out_ref.at[pl.ds(obase + qd * RQ, A), :],
                  xsend.at[NC], xrecv.at[NC], xp)
        xe.start()

        copies = []

        def out_copy(src_buf, src_off, rows, dst_row, sem_idx):
            cp = pltpu.make_async_copy(
                src_buf.at[pl.ds(src_off, rows), :],
                out_ref.at[pl.ds(dst_row, rows), :],
                csem.at[sem_idx],
            )
            cp.start()
            copies.append(cp)

        yd = []
        zd = []
        stage2 = []

        def drain_in(j):
            joff, jrows = CHUNKS[j]
            zd[j].wait_recv()
            if j in Y2P:
                p_off, p_rows, sidx = Y2P[j]
                p = rdma(zrbuf.at[pl.ds(p_off, p_rows), :],
                         out_ref.at[pl.ds(rbase + qz * RQ + p_off,
                                          p_rows), :],
                         ysend.at[sidx], yrecv.at[sidx], yp)
                p.start()
                stage2.append(p)
            out_copy(zrbuf, joff, jrows, rbase + qz * RQ + joff, NC + j)
            yd[j].wait_recv()
            if j in Z2P:
                p_off, p_rows, sidx = Z2P[j]
                p = rdma(yrbuf.at[pl.ds(p_off, p_rows), :],
                         out_ref.at[pl.ds(rbase + qy * RQ + p_off,
                                          p_rows), :],
                         zsend.at[sidx], zrecv.at[sidx], zp)
                p.start()
                stage2.append(p)
            out_copy(yrbuf, joff, jrows, rbase + qy * RQ + joff,
                     2 * NC + j)

        for k, (off, rows) in enumerate(CHUNKS):
            xd[k].wait_recv()
            dy = rdma(xrbuf.at[pl.ds(off, rows), :],
                      yrbuf.at[pl.ds(off, rows), :],
                      ysend.at[k], yrecv.at[k], yp)
            dy.start()
            yd.append(dy)
            dz = rdma(xrbuf.at[pl.ds(off, rows), :],
                      zrbuf.at[pl.ds(off, rows), :],
                      zsend.at[k], zrecv.at[k], zp)
            dz.start()
            zd.append(dz)
            out_copy(xrbuf, off, rows, rbase + q * RQ + off, k)
            if k >= 1:
                drain_in(k - 1)
        drain_in(NC - 1)

        keep_in.wait()
        keep_out = pltpu.make_async_copy(
            keep_buf, out_ref.at[pl.ds(my_x * m, m), :], keep_out_sem,
        )
        keep_out.start()

        xe.wait_recv()
        for p in stage2:
            p.wait_recv()
        for d in xd + yd + zd + stage2:
            d.wait_send()
        xe.wait_send()
        for cp in copies:
            cp.wait()
        keep_out.wait()

    n_y = NC + len(Y2P)
    n_z = NC + len(Z2P)
    return pl.pallas_call(
        body,
        out_shape=jax.ShapeDtypeStruct((out_m, half), x.dtype),
        in_specs=[pl.BlockSpec(memory_space=pl.ANY)],
        out_specs=pl.BlockSpec(memory_space=pl.ANY),
        scratch_shapes=[
            pltpu.VMEM((RQ + A, half), x.dtype),
            pltpu.VMEM((RQ, half), x.dtype),
            pltpu.VMEM((RQ, half), x.dtype),
            pltpu.VMEM((RQ, half), x.dtype),
            pltpu.VMEM((m, half), x.dtype),
            pltpu.SemaphoreType.DMA,
            pltpu.SemaphoreType.DMA,
            pltpu.SemaphoreType.DMA((2,)),
            pltpu.SemaphoreType.DMA((NC + 1,)),
            pltpu.SemaphoreType.DMA((NC + 1,)),
            pltpu.SemaphoreType.DMA((n_y,)),
            pltpu.SemaphoreType.DMA((n_y,)),
            pltpu.SemaphoreType.DMA((n_z,)),
            pltpu.SemaphoreType.DMA((n_z,)),
            pltpu.SemaphoreType.DMA((3 * NC,)),
        ],
        compiler_params=pltpu.CompilerParams(
            collective_id=0, vmem_limit_bytes=64 * 1024 * 1024,
        ),
    )(x)


# device time: 108480 ns/iter; 1.0051x vs baseline; 1.0051x over previous
import jax
import jax.numpy as jnp
from jax import lax
from jax.experimental import pallas as pl
from jax.experimental.pallas import tpu as pltpu

RQ = 1024
CHUNKS = ((0, 64), (64, 192), (256, 256), (512, 256), (768, 256))
NC = len(CHUNKS)
B = 344
C2 = 336
A = RQ - B - C2
XO = B + C2

def _pieces(lo, hi):
    out = {}
    idx = NC
    for k, (off, rows) in enumerate(CHUNKS):
        p_lo, p_hi = max(off, lo), min(off + rows, hi)
        if p_lo < p_hi:
            out[k] = (p_lo, p_hi - p_lo, idx)
            idx += 1
    return out

Y2P = _pieces(0, B)
Z2P = _pieces(B, B + C2)


def kernel(x):
    m, n = x.shape
    half = n // 2
    out_m = 2 * m

    def body(x_ref, out_ref, sbuf, xrbuf, yrbuf, zrbuf, keep_buf,
             keep_sem, keep_out_sem, stage_sems,
             xsend, xrecv, ysend, yrecv, zsend, zrecv, csem):
        my_x = lax.axis_index("x")
        my_y = lax.axis_index("y")
        my_z = lax.axis_index("z")
        xp = (1 - my_x, my_y, my_z)
        yp = (my_x, 1 - my_y, my_z)
        zp = (my_x, my_y, 1 - my_z)
        q = 2 * my_y + my_z
        qy = 2 * (1 - my_y) + my_z
        qz = 2 * my_y + (1 - my_z)
        qd = 2 * (1 - my_y) + (1 - my_z)

        ocol = (1 - my_x) * half
        obase = my_x * m
        rbase = (1 - my_x) * m

        def rdma(src, dst, ssem, rsem, dev):
            return pltpu.make_async_remote_copy(
                src_ref=src, dst_ref=dst, send_sem=ssem, recv_sem=rsem,
                device_id=dev, device_id_type=pl.DeviceIdType.MESH,
            )

        barrier = pltpu.get_barrier_semaphore()
        for nb in (xp, yp, zp):
            pl.semaphore_signal(
                barrier, inc=1, device_id=nb,
                device_id_type=pl.DeviceIdType.MESH,
            )
        pl.semaphore_wait(barrier, 3)

        keep_in = pltpu.make_async_copy(
            x_ref.at[:, pl.ds(my_x * half, half)], keep_buf, keep_sem,
        )
        keep_in.start()

        st0 = pltpu.make_async_copy(
            x_ref.at[pl.ds(q * RQ, RQ), pl.ds(ocol, half)],
            sbuf.at[pl.ds(0, RQ), :], stage_sems.at[0],
        )
        st1 = pltpu.make_async_copy(
            x_ref.at[pl.ds(qd * RQ + XO, A), pl.ds(ocol, half)],
            sbuf.at[pl.ds(RQ, A), :], stage_sems.at[1],
        )
        st0.start()
        st1.start()
        st0.wait()

        xd = []
        for k, (off, rows) in enumerate(CHUNKS):
            d = rdma(sbuf.at[pl.ds(off, rows), :],
                     xrbuf.at[pl.ds(off, rows), :],
                     xsend.at[k], xrecv.at[k], xp)
            d.start()
            xd.append(d)
        st1.wait()
        xe = rdma(sbuf.at[pl.ds(RQ, A), :],
                  out_ref.at[pl.ds(obase + qd * RQ + XO, A), :],
                  xsend.at[NC], xrecv.at[NC], xp)
        xe.start()

        copies = []

        def out_copy(src_buf, src_off, rows, dst_row, sem_idx):
            cp = pltpu.make_async_copy(
                src_buf.at[pl.ds(src_off, rows), :],
                out_ref.at[pl.ds(dst_row, rows), :],
                csem.at[sem_idx],
            )
            cp.start()
            copies.append(cp)

        yd = []
        zd = []
        stage2 = []

        def drain_in(j):
            joff, jrows = CHUNKS[j]
            zd[j].wait_recv()
            if j in Y2P:
                p_off, p_rows, sidx = Y2P[j]
                p = rdma(zrbuf.at[pl.ds(p_off, p_rows), :],
                         out_ref.at[pl.ds(rbase + qz * RQ + p_off,
                                          p_rows), :],
                         ysend.at[sidx], yrecv.at[sidx], yp)
                p.start()
                stage2.append(p)
            out_copy(zrbuf, joff, jrows, rbase + qz * RQ + joff, NC + j)
            yd[j].wait_recv()
            if j in Z2P:
                p_off, p_rows, sidx = Z2P[j]
                p = rdma(yrbuf.at[pl.ds(p_off, p_rows), :],
                         out_ref.at[pl.ds(rbase + qy * RQ + p_off,
                                          p_rows), :],
                         zsend.at[sidx], zrecv.at[sidx], zp)
                p.start()
                stage2.append(p)
            out_copy(yrbuf, joff, jrows, rbase + qy * RQ + joff,
                     2 * NC + j)

        for k, (off, rows) in enumerate(CHUNKS):
            xd[k].wait_recv()
            dy = rdma(xrbuf.at[pl.ds(off, rows), :],
                      yrbuf.at[pl.ds(off, rows), :],
                      ysend.at[k], yrecv.at[k], yp)
            dy.start()
            yd.append(dy)
            dz = rdma(xrbuf.at[pl.ds(off, rows), :],
                      zrbuf.at[pl.ds(off, rows), :],
                      zsend.at[k], zrecv.at[k], zp)
            dz.start()
            zd.append(dz)
            out_copy(xrbuf, off, rows, rbase + q * RQ + off, k)
            if k >= 1:
                drain_in(k - 1)
        drain_in(NC - 1)

        keep_in.wait()
        keep_out = pltpu.make_async_copy(
            keep_buf, out_ref.at[pl.ds(my_x * m, m), :], keep_out_sem,
        )
        keep_out.start()

        xe.wait_recv()
        for p in stage2:
            p.wait_recv()
        for d in xd + yd + zd + stage2:
            d.wait_send()
        xe.wait_send()
        for cp in copies:
            cp.wait()
        keep_out.wait()

    n_y = NC + len(Y2P)
    n_z = NC + len(Z2P)
    return pl.pallas_call(
        body,
        out_shape=jax.ShapeDtypeStruct((out_m, half), x.dtype),
        in_specs=[pl.BlockSpec(memory_space=pl.ANY)],
        out_specs=pl.BlockSpec(memory_space=pl.ANY),
        scratch_shapes=[
            pltpu.VMEM((RQ + A, half), x.dtype),
            pltpu.VMEM((RQ, half), x.dtype),
            pltpu.VMEM((RQ, half), x.dtype),
            pltpu.VMEM((RQ, half), x.dtype),
            pltpu.VMEM((m, half), x.dtype),
            pltpu.SemaphoreType.DMA,
            pltpu.SemaphoreType.DMA,
            pltpu.SemaphoreType.DMA((2,)),
            pltpu.SemaphoreType.DMA((NC + 1,)),
            pltpu.SemaphoreType.DMA((NC + 1,)),
            pltpu.SemaphoreType.DMA((n_y,)),
            pltpu.SemaphoreType.DMA((n_y,)),
            pltpu.SemaphoreType.DMA((n_z,)),
            pltpu.SemaphoreType.DMA((n_z,)),
            pltpu.SemaphoreType.DMA((3 * NC,)),
        ],
        compiler_params=pltpu.CompilerParams(
            collective_id=0, vmem_limit_bytes=64 * 1024 * 1024,
        ),
    )(x)


# device time: 104838 ns/iter; 1.0401x vs baseline; 1.0347x over previous
import jax
import jax.numpy as jnp
from jax import lax
from jax.experimental import pallas as pl
from jax.experimental.pallas import tpu as pltpu

RQ = 1024
CHUNKS = ((0, 64), (64, 192), (256, 256), (512, 256), (768, 256))
NC = len(CHUNKS)
A = 400
B = 312
C2 = RQ - A - B


def kernel(x):
    m, n = x.shape
    half = n // 2
    out_m = 2 * m

    def body(x_ref, out_ref, sbuf, xrbuf, yrbuf, zrbuf, keep_buf,
             keep_sem, keep_out_sem, stage_sems,
             xsend, xrecv, ysend, yrecv, zsend, zrecv, csem):
        my_x = lax.axis_index("x")
        my_y = lax.axis_index("y")
        my_z = lax.axis_index("z")
        xp = (1 - my_x, my_y, my_z)
        yp = (my_x, 1 - my_y, my_z)
        zp = (my_x, my_y, 1 - my_z)
        q = 2 * my_y + my_z
        qy = 2 * (1 - my_y) + my_z
        qz = 2 * my_y + (1 - my_z)
        qd = 2 * (1 - my_y) + (1 - my_z)

        ocol = (1 - my_x) * half
        obase = my_x * m
        rbase = (1 - my_x) * m

        def rdma(src, dst, ssem, rsem, dev):
            return pltpu.make_async_remote_copy(
                src_ref=src, dst_ref=dst, send_sem=ssem, recv_sem=rsem,
                device_id=dev, device_id_type=pl.DeviceIdType.MESH,
            )

        barrier = pltpu.get_barrier_semaphore()
        for nb in (xp, yp, zp):
            pl.semaphore_signal(
                barrier, inc=1, device_id=nb,
                device_id_type=pl.DeviceIdType.MESH,
            )
        pl.semaphore_wait(barrier, 3)

        keep_in = pltpu.make_async_copy(
            x_ref.at[:, pl.ds(my_x * half, half)], keep_buf, keep_sem,
        )
        keep_in.start()

        st0 = pltpu.make_async_copy(
            x_ref.at[pl.ds(q * RQ, RQ), pl.ds(ocol, half)],
            sbuf.at[pl.ds(0, RQ), :], stage_sems.at[0],
        )
        st1 = pltpu.make_async_copy(
            x_ref.at[pl.ds(qd * RQ, A), pl.ds(ocol, half)],
            sbuf.at[pl.ds(RQ, A), :], stage_sems.at[1],
        )
        st0.start()
        st1.start()
        st0.wait()

        xd = []
        for k, (off, rows) in enumerate(CHUNKS):
            d = rdma(sbuf.at[pl.ds(off, rows), :],
                     xrbuf.at[pl.ds(off, rows), :],
                     xsend.at[k], xrecv.at[k], xp)
            d.start()
            xd.append(d)
        st1.wait()
        xe = rdma(sbuf.at[pl.ds(RQ, A), :],
                  out_ref.at[pl.ds(obase + qd * RQ, A), :],
                  xsend.at[NC], xrecv.at[NC], xp)
        xe.start()

        yd = []
        zd = []
        copies = []
        for k, (off, rows) in enumerate(CHUNKS):
            xd[k].wait_recv()
            dy = rdma(xrbuf.at[pl.ds(off, rows), :],
                      yrbuf.at[pl.ds(off, rows), :],
                      ysend.at[k], yrecv.at[k], yp)
            dy.start()
            yd.append(dy)
            dz = rdma(xrbuf.at[pl.ds(off, rows), :],
                      zrbuf.at[pl.ds(off, rows), :],
                      zsend.at[k], zrecv.at[k], zp)
            dz.start()
            zd.append(dz)
            cp = pltpu.make_async_copy(
                xrbuf.at[pl.ds(off, rows), :],
                out_ref.at[pl.ds(rbase + q * RQ + off, rows), :],
                csem.at[k],
            )
            cp.start()
            copies.append(cp)

        for k in range(NC - 1):
            off, rows = CHUNKS[k]
            zd[k].wait_recv()
            cp = pltpu.make_async_copy(
                zrbuf.at[pl.ds(off, rows), :],
                out_ref.at[pl.ds(rbase + qz * RQ + off, rows), :],
                csem.at[NC + k],
            )
            cp.start()
            copies.append(cp)
        y2 = rdma(zrbuf.at[pl.ds(A, B), :],
                  out_ref.at[pl.ds(rbase + qz * RQ + A, B), :],
                  ysend.at[NC], yrecv.at[NC], yp)
        y2.start()

        for k, (off, rows) in enumerate(CHUNKS):
            yd[k].wait_recv()
            cp = pltpu.make_async_copy(
                yrbuf.at[pl.ds(off, rows), :],
                out_ref.at[pl.ds(rbase + qy * RQ + off, rows), :],
                csem.at[2 * NC + k],
            )
            cp.start()
            copies.append(cp)
        z2 = rdma(yrbuf.at[pl.ds(A + B, C2), :],
                  out_ref.at[pl.ds(rbase + qy * RQ + A + B, C2), :],
                  zsend.at[NC], zrecv.at[NC], zp)
        z2.start()

        off, rows = CHUNKS[NC - 1]
        zd[NC - 1].wait_recv()
        cp = pltpu.make_async_copy(
            zrbuf.at[pl.ds(off, rows), :],
            out_ref.at[pl.ds(rbase + qz * RQ + off, rows), :],
            csem.at[NC + NC - 1],
        )
        cp.start()
        copies.append(cp)

        keep_in.wait()
        keep_out = pltpu.make_async_copy(
            keep_buf, out_ref.at[pl.ds(my_x * m, m), :], keep_out_sem,
        )
        keep_out.start()

        xe.wait_recv()
        y2.wait_recv()
        z2.wait_recv()
        for d in xd + yd + zd:
            d.wait_send()
        xe.wait_send()
        y2.wait_send()
        z2.wait_send()
        for cp in copies:
            cp.wait()
        keep_out.wait()

    return pl.pallas_call(
        body,
        out_shape=jax.ShapeDtypeStruct((out_m, half), x.dtype),
        in_specs=[pl.BlockSpec(memory_space=pl.ANY)],
        out_specs=pl.BlockSpec(memory_space=pl.ANY),
        scratch_shapes=[
            pltpu.VMEM((RQ + A, half), x.dtype),
            pltpu.VMEM((RQ, half), x.dtype),
            pltpu.VMEM((RQ, half), x.dtype),
            pltpu.VMEM((RQ, half), x.dtype),
            pltpu.VMEM((m, half), x.dtype),
            pltpu.SemaphoreType.DMA,
            pltpu.SemaphoreType.DMA,
            pltpu.SemaphoreType.DMA((2,)),
            pltpu.SemaphoreType.DMA((NC + 1,)),
            pltpu.SemaphoreType.DMA((NC + 1,)),
            pltpu.SemaphoreType.DMA((NC + 1,)),
            pltpu.SemaphoreType.DMA((NC + 1,)),
            pltpu.SemaphoreType.DMA((NC + 1,)),
            pltpu.SemaphoreType.DMA((NC + 1,)),
            pltpu.SemaphoreType.DMA((3 * NC,)),
        ],
        compiler_params=pltpu.CompilerParams(
            collective_id=0, vmem_limit_bytes=64 * 1024 * 1024,
        ),
    )(x)
